# baseline (speedup 1.0000x reference)
import functools

import jax
import jax.numpy as jnp
from jax import lax
from jax.experimental import pallas as pl
from jax.experimental.pallas import tpu as pltpu
from jax.experimental.pallas import tpu_sc as plsc

M0, M1, M2, M3 = 64, 16, 8, 4096
TOTAL = M0 * M1 * M2 * M3
FACE = M2 * M3
NE = 524288
NC = 2
NS = 16
NW = NC * NS
RANGE = TOTAL // NW
NF = RANGE // FACE
CH = 2048
NCH = NE // CH
CAP = 27000
SENT = jnp.int32(2 ** 28)

_mesh = plsc.VectorSubcoreMesh(core_axis_name="c", subcore_axis_name="s")


@functools.partial(
    pl.kernel,
    mesh=_mesh,
    out_type=jax.ShapeDtypeStruct((M0, M1, M2, M3), jnp.float32),
    compiler_params=pltpu.CompilerParams(
        needs_layout_passes=False, use_tc_tiling_on_sc=True),
    scratch_types=[
        pltpu.VMEM((M2, M3), jnp.float32),
        pltpu.VMEM((M2, M3), jnp.float32),
        pltpu.VMEM((CAP + 32,), jnp.int32),
        pltpu.VMEM((CAP + 32,), jnp.float32),
        pltpu.VMEM((CH,), jnp.int32),
        pltpu.VMEM((CH,), jnp.float32),
        pltpu.VMEM((CH,), jnp.int32),
        pltpu.VMEM((CH,), jnp.float32),
        pltpu.SemaphoreType.DMA,
        pltpu.SemaphoreType.DMA,
        pltpu.SemaphoreType.DMA,
        pltpu.SemaphoreType.DMA,
    ],
)
def _scatter_add(mem_hbm, idx_hbm, val_hbm, out_hbm,
                 bufA, bufB, rel_l, val_l, idx_sA, val_sA, idx_sB, val_sB,
                 semAi, semAv, semBi, semBv):
    wid = lax.axis_index("s") * NC + lax.axis_index("c")
    lo = wid * RANGE
    lane = lax.iota(jnp.int32, 16)
    ones = jnp.ones((16,), jnp.int32)
    u32 = jnp.uint32

    def scan_chunk(stg_i, stg_v, wm):
        def vec_body(j, wm):
            i16 = stg_i[pl.ds(j * 16, 16)]
            v16 = stg_v[pl.ds(j * 16, 16)]
            rel = i16 - lo
            m = lax.bitcast_convert_type(rel, u32) < u32(RANGE)
            ck = plsc.cumsum(ones, mask=m)
            pos = jnp.minimum(wm + ck, CAP + 15)
            plsc.store_scatter(rel_l, [pos], rel, mask=m)
            plsc.store_scatter(val_l, [pos], v16, mask=m)
            return wm + plsc.all_reduce_population_count(m)

        return plsc.parallel_loop(0, CH // 16, unroll=16, carry=wm)(vec_body)

    def pair_body(c2, wm):
        ca = 2 * c2
        cb = 2 * c2 + 1
        pltpu.make_async_copy(idx_hbm.at[pl.ds(ca * CH, CH)], idx_sA, semAi).wait()
        pltpu.make_async_copy(val_hbm.at[pl.ds(ca * CH, CH)], val_sA, semAv).wait()
        wm = scan_chunk(idx_sA, val_sA, wm)
        na = jnp.minimum(ca + 2, NCH - 2)
        pltpu.async_copy(idx_hbm.at[pl.ds(na * CH, CH)], idx_sA, semAi)
        pltpu.async_copy(val_hbm.at[pl.ds(na * CH, CH)], val_sA, semAv)
        pltpu.make_async_copy(idx_hbm.at[pl.ds(cb * CH, CH)], idx_sB, semBi).wait()
        pltpu.make_async_copy(val_hbm.at[pl.ds(cb * CH, CH)], val_sB, semBv).wait()
        wm = scan_chunk(idx_sB, val_sB, wm)
        nb = jnp.minimum(cb + 2, NCH - 1)
        pltpu.async_copy(idx_hbm.at[pl.ds(nb * CH, CH)], idx_sB, semBi)
        pltpu.async_copy(val_hbm.at[pl.ds(nb * CH, CH)], val_sB, semBv)
        return wm

    pltpu.async_copy(idx_hbm.at[pl.ds(0, CH)], idx_sA, semAi)
    pltpu.async_copy(val_hbm.at[pl.ds(0, CH)], val_sA, semAv)
    pltpu.async_copy(idx_hbm.at[pl.ds(CH, CH)], idx_sB, semBi)
    pltpu.async_copy(val_hbm.at[pl.ds(CH, CH)], val_sB, semBv)

    wm = lax.fori_loop(0, NCH // 2, pair_body, jnp.full((16,), -1, jnp.int32))

    pltpu.make_async_copy(idx_hbm.at[pl.ds((NCH - 2) * CH, CH)], idx_sA, semAi).wait()
    pltpu.make_async_copy(val_hbm.at[pl.ds((NCH - 2) * CH, CH)], val_sA, semAv).wait()
    pltpu.make_async_copy(idx_hbm.at[pl.ds((NCH - 1) * CH, CH)], idx_sB, semBi).wait()
    pltpu.make_async_copy(val_hbm.at[pl.ds((NCH - 1) * CH, CH)], val_sB, semBv).wait()

    wcl = jnp.minimum(wm + 1, CAP)
    plsc.store_scatter(rel_l, [wcl + lane], jnp.full((16,), SENT, jnp.int32))
    w = wcl[0]

    nj = (w + 15) // 16

    def blk_body(g, w):
        gfA = wid * NF + 2 * g
        gfB = gfA + 1
        miA, fiA = gfA // M1, gfA % M1
        miB, fiB = gfB // M1, gfB % M1
        pltpu.sync_copy(mem_hbm.at[miA, fiA], bufA)
        pltpu.sync_copy(mem_hbm.at[miB, fiB], bufB)
        blo = g * (2 * FACE)

        def apply(j):
            r16 = rel_l[pl.ds(j * 16, 16)]
            v16 = val_l[pl.ds(j * 16, 16)]
            relw = r16 - blo
            relu = lax.bitcast_convert_type(relw, u32)
            mA = relu < u32(FACE)
            mB = (relu < u32(2 * FACE)) ^ mA
            r12 = lax.shift_right_logical(relw, 12)
            i1 = relw & (M3 - 1)
            plsc.addupdate_scatter(bufA, [r12, i1], v16, mask=mA)
            plsc.addupdate_scatter(bufB, [r12 - M2, i1], v16, mask=mB)

        plsc.parallel_loop(0, nj, unroll=8)(apply)
        pltpu.sync_copy(bufA, out_hbm.at[miA, fiA])
        pltpu.sync_copy(bufB, out_hbm.at[miB, fiB])
        return w

    lax.fori_loop(0, NF // 2, blk_body, w)


def kernel(mem, idx, val):
    return _scatter_add(mem, idx, val)


# speedup vs baseline: 1.0558x; 1.0558x over previous
import functools

import jax
import jax.numpy as jnp
from jax import lax
from jax.experimental import pallas as pl
from jax.experimental.pallas import tpu as pltpu
from jax.experimental.pallas import tpu_sc as plsc

M0, M1, M2, M3 = 64, 16, 8, 4096
TOTAL = M0 * M1 * M2 * M3
FACE = M2 * M3
NE = 524288
NC = 2
NS = 16
NW = NC * NS
RANGE = TOTAL // NW
NF = RANGE // FACE
HALF = RANGE // 2
QRT = RANGE // 4
CH = 2048
NCH = NE // CH
CAP = 26992
LEND = CAP + 64
ARENA = FACE

_mesh = plsc.VectorSubcoreMesh(core_axis_name="c", subcore_axis_name="s")


@functools.partial(
    pl.kernel,
    mesh=_mesh,
    out_type=jax.ShapeDtypeStruct((M0, M1, M2, M3), jnp.float32),
    compiler_params=pltpu.CompilerParams(
        needs_layout_passes=False, use_tc_tiling_on_sc=True),
    scratch_types=[
        pltpu.VMEM((M2, M3), jnp.float32),
        pltpu.VMEM((M2, M3), jnp.float32),
        pltpu.VMEM((LEND,), jnp.int32),
        pltpu.VMEM((LEND,), jnp.float32),
        pltpu.VMEM((CH,), jnp.int32),
        pltpu.VMEM((CH,), jnp.float32),
        pltpu.VMEM((CH,), jnp.int32),
        pltpu.VMEM((CH,), jnp.float32),
        pltpu.SemaphoreType.DMA,
        pltpu.SemaphoreType.DMA,
        pltpu.SemaphoreType.DMA,
        pltpu.SemaphoreType.DMA,
    ],
)
def _scatter_add(mem_hbm, idx_hbm, val_hbm, out_hbm,
                 bufA, bufB, rel_l, val_l, idx_sA, val_sA, idx_sB, val_sB,
                 semAi, semAv, semBi, semBv):
    wid = lax.axis_index("s") * NC + lax.axis_index("c")
    lo = wid * RANGE
    lane = lax.iota(jnp.int32, 16)
    ones = jnp.ones((16,), jnp.int32)
    u32 = jnp.uint32
    f32 = jnp.float32
    i32 = jnp.int32

    def bc_u(x):
        return lax.bitcast_convert_type(x, u32)

    def scan_chunk(stg_i, stg_v, wm):
        def vec_body(j, wm):
            i16 = stg_i[pl.ds(j * 16, 16)]
            v16 = stg_v[pl.ds(j * 16, 16)]
            rel = i16 - lo
            m = bc_u(rel) < u32(RANGE)
            ck = plsc.cumsum(ones, mask=m)
            pos = jnp.minimum(wm + ck, CAP + 15)
            plsc.store_scatter(rel_l, [pos], rel, mask=m)
            plsc.store_scatter(val_l, [pos], v16, mask=m)
            return wm + plsc.all_reduce_population_count(m)

        return plsc.parallel_loop(0, CH // 16, unroll=16, carry=wm)(vec_body)

    def pair_body(c2, wm):
        ca = 2 * c2
        cb = 2 * c2 + 1
        pltpu.make_async_copy(idx_hbm.at[pl.ds(ca * CH, CH)], idx_sA, semAi).wait()
        pltpu.make_async_copy(val_hbm.at[pl.ds(ca * CH, CH)], val_sA, semAv).wait()
        wm = scan_chunk(idx_sA, val_sA, wm)
        na = jnp.minimum(ca + 2, NCH - 2)
        pltpu.async_copy(idx_hbm.at[pl.ds(na * CH, CH)], idx_sA, semAi)
        pltpu.async_copy(val_hbm.at[pl.ds(na * CH, CH)], val_sA, semAv)
        pltpu.make_async_copy(idx_hbm.at[pl.ds(cb * CH, CH)], idx_sB, semBi).wait()
        pltpu.make_async_copy(val_hbm.at[pl.ds(cb * CH, CH)], val_sB, semBv).wait()
        wm = scan_chunk(idx_sB, val_sB, wm)
        nb = jnp.minimum(cb + 2, NCH - 1)
        pltpu.async_copy(idx_hbm.at[pl.ds(nb * CH, CH)], idx_sB, semBi)
        pltpu.async_copy(val_hbm.at[pl.ds(nb * CH, CH)], val_sB, semBv)
        return wm

    pltpu.async_copy(idx_hbm.at[pl.ds(0, CH)], idx_sA, semAi)
    pltpu.async_copy(val_hbm.at[pl.ds(0, CH)], val_sA, semAv)
    pltpu.async_copy(idx_hbm.at[pl.ds(CH, CH)], idx_sB, semBi)
    pltpu.async_copy(val_hbm.at[pl.ds(CH, CH)], val_sB, semBv)

    wm = lax.fori_loop(0, NCH // 2, pair_body, jnp.full((16,), -1, jnp.int32))

    pltpu.make_async_copy(idx_hbm.at[pl.ds((NCH - 2) * CH, CH)], idx_sA, semAi).wait()
    pltpu.make_async_copy(val_hbm.at[pl.ds((NCH - 2) * CH, CH)], val_sA, semAv).wait()
    pltpu.make_async_copy(idx_hbm.at[pl.ds((NCH - 1) * CH, CH)], idx_sB, semBi).wait()
    pltpu.make_async_copy(val_hbm.at[pl.ds((NCH - 1) * CH, CH)], val_sB, semBv).wait()

    w = jnp.minimum(wm[0] + 1, CAP)

    nv0 = (w + 15) // 16

    def split1(j, carry):
        wlm, ehv = carry
        valid = (j * 16 + lane) < w
        r16 = rel_l[pl.ds(j * 16, 16)]
        v16 = val_l[pl.ds(j * 16, 16)]
        mL = valid & (bc_u(r16) < u32(HALF))
        mH = valid ^ mL
        ckL = plsc.cumsum(ones, mask=mL)
        posL = wlm + ckL
        plsc.store_scatter(bufA, [posL >> 12, posL & (M3 - 1)],
                           lax.bitcast_convert_type(r16, f32), mask=mL)
        plsc.store_scatter(bufB, [posL >> 12, posL & (M3 - 1)], v16, mask=mL)
        ckH = plsc.cumsum(ones, mask=mH)
        posH = ehv - ckH
        plsc.store_scatter(bufA, [posH >> 12, posH & (M3 - 1)],
                           lax.bitcast_convert_type(r16, f32), mask=mH)
        plsc.store_scatter(bufB, [posH >> 12, posH & (M3 - 1)], v16, mask=mH)
        return (wlm + plsc.all_reduce_population_count(mL),
                ehv - plsc.all_reduce_population_count(mH))

    wlm, ehv = plsc.parallel_loop(
        0, nv0, carry=(jnp.full((16,), -1, i32), jnp.full((16,), ARENA, i32)),
        unroll=4)(split1)
    wl = wlm[0] + 1
    eh = ehv[0]

    a1 = ((wl + 15) // 16) * 16

    def make_split2(lo_s, hi_s, qsplit):
        s0 = (lo_s // 16) * 16
        nv = (hi_s - s0 + 15) // 16

        def split2(j, carry):
            am, dv = carry
            off = s0 + j * 16
            p16 = off + lane
            valid = (p16 >= lo_s) & (p16 < hi_s)
            r = off >> 12
            c = off & (M3 - 1)
            r16 = lax.bitcast_convert_type(bufA[r, pl.ds(c, 16)], i32)
            v16 = bufB[r, pl.ds(c, 16)]
            m0 = valid & (bc_u(r16) < u32(qsplit))
            m1 = valid ^ m0
            ck0 = plsc.cumsum(ones, mask=m0)
            pos0 = am + ck0
            plsc.store_scatter(rel_l, [pos0], r16, mask=m0)
            plsc.store_scatter(val_l, [pos0], v16, mask=m0)
            ck1 = plsc.cumsum(ones, mask=m1)
            pos1 = dv - ck1
            plsc.store_scatter(rel_l, [pos1], r16, mask=m1)
            plsc.store_scatter(val_l, [pos1], v16, mask=m1)
            return (am + plsc.all_reduce_population_count(m0),
                    dv - plsc.all_reduce_population_count(m1))

        return nv, split2

    nvL, splitL = make_split2(jnp.int32(0), wl, QRT)
    amL, dvL = plsc.parallel_loop(
        0, nvL, carry=(jnp.full((16,), -1, i32), jnp.full((16,), a1, i32)),
        unroll=4)(splitL)
    q0c = amL[0] + 1
    q1s = dvL[0]

    nvH, splitH = make_split2(eh, jnp.int32(ARENA), HALF + QRT)
    amH, dvH = plsc.parallel_loop(
        0, nvH, carry=(jnp.full((16,), a1 - 1, i32), jnp.full((16,), LEND, i32)),
        unroll=4)(splitH)
    q2c = amH[0] + 1 - a1
    q3s = dvH[0]

    qlo = [jnp.int32(0), q1s, a1, q3s]
    qhi = [q0c, a1, a1 + q2c, jnp.int32(LEND)]

    for q in range(4):
        lo_q, hi_q = qlo[q], qhi[q]
        s_q = (lo_q // 16) * 16
        nv_q = (hi_q - s_q + 15) // 16

        def blk_body(t, _, q=q, lo_q=lo_q, hi_q=hi_q, s_q=s_q, nv_q=nv_q):
            g = q * 4 + t
            gfA = wid * NF + 2 * g
            gfB = gfA + 1
            miA, fiA = gfA // M1, gfA % M1
            miB, fiB = gfB // M1, gfB % M1
            pltpu.sync_copy(mem_hbm.at[miA, fiA], bufA)
            pltpu.sync_copy(mem_hbm.at[miB, fiB], bufB)
            blo = g * (2 * FACE)

            def apply(j):
                off = s_q + j * 16
                p16 = off + lane
                valid = (p16 >= lo_q) & (p16 < hi_q)
                r16 = rel_l[pl.ds(off, 16)]
                v16 = val_l[pl.ds(off, 16)]
                relw = r16 - blo
                relu = bc_u(relw)
                mA = valid & (relu < u32(FACE))
                mB = (valid & (relu < u32(2 * FACE))) ^ mA
                r12 = lax.shift_right_logical(relw, 12)
                i1 = relw & (M3 - 1)
                plsc.addupdate_scatter(bufA, [r12, i1], v16, mask=mA)
                plsc.addupdate_scatter(bufB, [r12 - M2, i1], v16, mask=mB)

            plsc.parallel_loop(0, nv_q, unroll=8)(apply)
            pltpu.sync_copy(bufA, out_hbm.at[miA, fiA])
            pltpu.sync_copy(bufB, out_hbm.at[miB, fiB])
            return 0

        lax.fori_loop(0, 4, blk_body, 0)


def kernel(mem, idx, val):
    return _scatter_add(mem, idx, val)


# speedup vs baseline: 1.1911x; 1.1282x over previous
import functools

import jax
import jax.numpy as jnp
from jax import lax
from jax.experimental import pallas as pl
from jax.experimental.pallas import tpu as pltpu
from jax.experimental.pallas import tpu_sc as plsc

M0, M1, M2, M3 = 64, 16, 8, 4096
TOTAL = M0 * M1 * M2 * M3
FACE = M2 * M3
NE = 524288
NC = 2
NS = 16
NW = NC * NS
RANGE = TOTAL // NW
NF = RANGE // FACE
HALF = RANGE // 2
QRT = RANGE // 4
CH = 4096
NCH = NE // CH
CAP = 22896
LEND = CAP + 64
ARENA = FACE

_mesh = plsc.VectorSubcoreMesh(core_axis_name="c", subcore_axis_name="s")


@functools.partial(
    pl.kernel,
    mesh=_mesh,
    out_type=jax.ShapeDtypeStruct((M0, M1, M2, M3), jnp.float32),
    compiler_params=pltpu.CompilerParams(
        needs_layout_passes=False, use_tc_tiling_on_sc=True),
    scratch_types=[
        pltpu.VMEM((M2, M3), jnp.float32),
        pltpu.VMEM((M2, M3), jnp.float32),
        pltpu.VMEM((LEND,), jnp.int32),
        pltpu.VMEM((LEND,), jnp.float32),
        pltpu.VMEM((CH,), jnp.int32),
        pltpu.VMEM((CH,), jnp.float32),
        pltpu.VMEM((CH,), jnp.int32),
        pltpu.VMEM((CH,), jnp.float32),
        pltpu.SemaphoreType.DMA,
        pltpu.SemaphoreType.DMA,
        pltpu.SemaphoreType.DMA,
        pltpu.SemaphoreType.DMA,
    ],
)
def _scatter_add(mem_hbm, idx_hbm, val_hbm, out_hbm,
                 bufA, bufB, rel_l, val_l, idx_sA, val_sA, idx_sB, val_sB,
                 semAi, semAv, semBi, semBv):
    wid = lax.axis_index("s") * NC + lax.axis_index("c")
    lo = wid * RANGE
    lane = lax.iota(jnp.int32, 16)
    ones = jnp.ones((16,), jnp.int32)
    u32 = jnp.uint32
    f32 = jnp.float32
    i32 = jnp.int32

    def bc_u(x):
        return lax.bitcast_convert_type(x, u32)

    def scan_chunk(stg_i, stg_v, wm):
        def vec_body(j, wm):
            i16 = stg_i[pl.ds(j * 16, 16)]
            v16 = stg_v[pl.ds(j * 16, 16)]
            rel = i16 - lo
            m = bc_u(rel) < u32(RANGE)
            ck = plsc.cumsum(ones, mask=m)
            pos = jnp.minimum(wm + ck, CAP + 15)
            plsc.store_scatter(rel_l, [pos], rel, mask=m)
            plsc.store_scatter(val_l, [pos], v16, mask=m)
            return wm + plsc.all_reduce_population_count(m)

        return plsc.parallel_loop(0, CH // 16, unroll=16, carry=wm)(vec_body)

    def pair_body(c2, wm):
        ca = 2 * c2
        cb = 2 * c2 + 1
        pltpu.make_async_copy(idx_hbm.at[pl.ds(ca * CH, CH)], idx_sA, semAi).wait()
        pltpu.make_async_copy(val_hbm.at[pl.ds(ca * CH, CH)], val_sA, semAv).wait()
        wm = scan_chunk(idx_sA, val_sA, wm)
        na = jnp.minimum(ca + 2, NCH - 2)
        pltpu.async_copy(idx_hbm.at[pl.ds(na * CH, CH)], idx_sA, semAi)
        pltpu.async_copy(val_hbm.at[pl.ds(na * CH, CH)], val_sA, semAv)
        pltpu.make_async_copy(idx_hbm.at[pl.ds(cb * CH, CH)], idx_sB, semBi).wait()
        pltpu.make_async_copy(val_hbm.at[pl.ds(cb * CH, CH)], val_sB, semBv).wait()
        wm = scan_chunk(idx_sB, val_sB, wm)
        nb = jnp.minimum(cb + 2, NCH - 1)
        pltpu.async_copy(idx_hbm.at[pl.ds(nb * CH, CH)], idx_sB, semBi)
        pltpu.async_copy(val_hbm.at[pl.ds(nb * CH, CH)], val_sB, semBv)
        return wm

    pltpu.async_copy(idx_hbm.at[pl.ds(0, CH)], idx_sA, semAi)
    pltpu.async_copy(val_hbm.at[pl.ds(0, CH)], val_sA, semAv)
    pltpu.async_copy(idx_hbm.at[pl.ds(CH, CH)], idx_sB, semBi)
    pltpu.async_copy(val_hbm.at[pl.ds(CH, CH)], val_sB, semBv)

    wm = lax.fori_loop(0, NCH // 2, pair_body, jnp.full((16,), -1, jnp.int32))

    pltpu.make_async_copy(idx_hbm.at[pl.ds((NCH - 2) * CH, CH)], idx_sA, semAi).wait()
    pltpu.make_async_copy(val_hbm.at[pl.ds((NCH - 2) * CH, CH)], val_sA, semAv).wait()
    pltpu.make_async_copy(idx_hbm.at[pl.ds((NCH - 1) * CH, CH)], idx_sB, semBi).wait()
    pltpu.make_async_copy(val_hbm.at[pl.ds((NCH - 1) * CH, CH)], val_sB, semBv).wait()

    w = jnp.minimum(wm[0] + 1, CAP)

    nv0 = (w + 15) // 16

    def split1(j, carry):
        wlm, ehv = carry
        valid = (j * 16 + lane) < w
        r16 = rel_l[pl.ds(j * 16, 16)]
        v16 = val_l[pl.ds(j * 16, 16)]
        mL = valid & (bc_u(r16) < u32(HALF))
        mH = valid ^ mL
        ckL = plsc.cumsum(ones, mask=mL)
        posL = wlm + ckL
        plsc.store_scatter(bufA, [posL >> 12, posL & (M3 - 1)],
                           lax.bitcast_convert_type(r16, f32), mask=mL)
        plsc.store_scatter(bufB, [posL >> 12, posL & (M3 - 1)], v16, mask=mL)
        ckH = plsc.cumsum(ones, mask=mH)
        posH = ehv - ckH
        plsc.store_scatter(bufA, [posH >> 12, posH & (M3 - 1)],
                           lax.bitcast_convert_type(r16, f32), mask=mH)
        plsc.store_scatter(bufB, [posH >> 12, posH & (M3 - 1)], v16, mask=mH)
        return (wlm + plsc.all_reduce_population_count(mL),
                ehv - plsc.all_reduce_population_count(mH))

    wlm, ehv = plsc.parallel_loop(
        0, nv0, carry=(jnp.full((16,), -1, i32), jnp.full((16,), ARENA, i32)),
        unroll=4)(split1)
    wl = wlm[0] + 1
    eh = ehv[0]

    a1 = ((wl + 15) // 16) * 16

    def make_split2(lo_s, hi_s, qsplit):
        s0 = (lo_s // 16) * 16
        nv = (hi_s - s0 + 15) // 16

        def split2(j, carry):
            am, dv = carry
            off = s0 + j * 16
            p16 = off + lane
            valid = (p16 >= lo_s) & (p16 < hi_s)
            r = off >> 12
            c = off & (M3 - 1)
            r16 = lax.bitcast_convert_type(bufA[r, pl.ds(c, 16)], i32)
            v16 = bufB[r, pl.ds(c, 16)]
            m0 = valid & (bc_u(r16) < u32(qsplit))
            m1 = valid ^ m0
            ck0 = plsc.cumsum(ones, mask=m0)
            pos0 = am + ck0
            plsc.store_scatter(rel_l, [pos0], r16, mask=m0)
            plsc.store_scatter(val_l, [pos0], v16, mask=m0)
            ck1 = plsc.cumsum(ones, mask=m1)
            pos1 = dv - ck1
            plsc.store_scatter(rel_l, [pos1], r16, mask=m1)
            plsc.store_scatter(val_l, [pos1], v16, mask=m1)
            return (am + plsc.all_reduce_population_count(m0),
                    dv - plsc.all_reduce_population_count(m1))

        return nv, split2

    nvL, splitL = make_split2(jnp.int32(0), wl, QRT)
    amL, dvL = plsc.parallel_loop(
        0, nvL, carry=(jnp.full((16,), -1, i32), jnp.full((16,), a1, i32)),
        unroll=4)(splitL)
    q0c = amL[0] + 1
    q1s = dvL[0]

    nvH, splitH = make_split2(eh, jnp.int32(ARENA), HALF + QRT)
    amH, dvH = plsc.parallel_loop(
        0, nvH, carry=(jnp.full((16,), a1 - 1, i32), jnp.full((16,), LEND, i32)),
        unroll=4)(splitH)
    q2c = amH[0] + 1 - a1
    q3s = dvH[0]

    qlo = [jnp.int32(0), q1s, a1, q3s]
    qhi = [q0c, a1, a1 + q2c, jnp.int32(LEND)]

    for q in range(4):
        lo_q, hi_q = qlo[q], qhi[q]
        s_q = (lo_q // 16) * 16
        nv_q = (hi_q - s_q + 15) // 16

        def blk_body(t, _, q=q, lo_q=lo_q, hi_q=hi_q, s_q=s_q, nv_q=nv_q):
            g = q * 4 + t
            gfA = wid * NF + 2 * g
            gfB = gfA + 1
            miA, fiA = gfA // M1, gfA % M1
            miB, fiB = gfB // M1, gfB % M1
            cpA = pltpu.async_copy(mem_hbm.at[miA, fiA], bufA, semAi)
            cpB = pltpu.async_copy(mem_hbm.at[miB, fiB], bufB, semBi)
            cpA.wait()
            cpB.wait()
            blo = g * (2 * FACE)

            def apply(j):
                off = s_q + j * 16
                p16 = off + lane
                valid = (p16 >= lo_q) & (p16 < hi_q)
                r16 = rel_l[pl.ds(off, 16)]
                v16 = val_l[pl.ds(off, 16)]
                relw = r16 - blo
                relu = bc_u(relw)
                mA = valid & (relu < u32(FACE))
                mB = (valid & (relu < u32(2 * FACE))) ^ mA
                r12 = lax.shift_right_logical(relw, 12)
                i1 = relw & (M3 - 1)
                plsc.addupdate_scatter(bufA, [r12, i1], v16, mask=mA)
                plsc.addupdate_scatter(bufB, [r12 - M2, i1], v16, mask=mB)

            plsc.parallel_loop(0, nv_q, unroll=8)(apply)
            oA = pltpu.async_copy(bufA, out_hbm.at[miA, fiA], semAv)
            oB = pltpu.async_copy(bufB, out_hbm.at[miB, fiB], semBv)
            oA.wait()
            oB.wait()
            return 0

        lax.fori_loop(0, 4, blk_body, 0)


def kernel(mem, idx, val):
    return _scatter_add(mem, idx, val)
